# Initial kernel scaffold; baseline (speedup 1.0000x reference)
import jax
import jax.numpy as jnp
from jax.experimental import pallas as pl


def kernel(point_cloud, point_cloud_features, camera_intrinsics, T_pointcloud_camera, camera_width, camera_height):
    raise NotImplementedError("write your pallas kernel here")


# speedup vs baseline: 9.4453x; 9.4453x over previous
import functools

import jax
import jax.numpy as jnp
from jax import lax
from jax.experimental import pallas as pl
from jax.experimental.pallas import tpu as pltpu

_NEAR_PLANE = 0.8
_FAR_PLANE = 1000.0
_DEPTH_TO_SORT_KEY_SCALE = 100.0
_KEY_DEPTH_BITS = 17
_INTERPRET = False

_C = 512
_RB = 64


def _bf16(v):
    return v.astype(jnp.bfloat16).astype(jnp.float32)


def _attrs_key_body(params_ref, pc_ref, ft_ref, attrs_ref, key_ref):
    p = params_ref
    xc = pc_ref[0]
    yc = pc_ref[1]
    d = pc_ref[2]
    fx = p[12]
    fy = p[13]
    cx = p[14]
    cy = p[15]
    width = p[16]
    height = p[17]
    tiles_per_row = p[18]
    d_safe = jnp.where(jnp.abs(d) > 1e-6, d, 1e-6)
    u = fx * xc / d_safe + cx
    v = fy * yc / d_safe + cy
    mask = ((d > _NEAR_PLANE) & (d < _FAR_PLANE)
            & (u >= 0) & (u < width) & (v >= 0) & (v < height))

    qx = ft_ref[0]
    qy = ft_ref[1]
    qz = ft_ref[2]
    qw = ft_ref[3]
    inv_qn = 1.0 / (jnp.sqrt(qx * qx + qy * qy + qz * qz + qw * qw) + 1e-8)
    qx = qx * inv_qn
    qy = qy * inv_qn
    qz = qz * inv_qn
    qw = qw * inv_qn
    r00 = 1.0 - 2.0 * (qy * qy + qz * qz)
    r01 = 2.0 * (qx * qy - qw * qz)
    r02 = 2.0 * (qx * qz + qw * qy)
    r10 = 2.0 * (qx * qy + qw * qz)
    r11 = 1.0 - 2.0 * (qx * qx + qz * qz)
    r12 = 2.0 * (qy * qz - qw * qx)
    r20 = 2.0 * (qx * qz - qw * qy)
    r21 = 2.0 * (qy * qz + qw * qx)
    r22 = 1.0 - 2.0 * (qx * qx + qy * qy)
    s0 = jnp.exp(ft_ref[4])
    s1 = jnp.exp(ft_ref[5])
    s2 = jnp.exp(ft_ref[6])
    m00 = r00 * s0
    m01 = r01 * s1
    m02 = r02 * s2
    m10 = r10 * s0
    m11 = r11 * s1
    m12 = r12 * s2
    m20 = r20 * s0
    m21 = r21 * s1
    m22 = r22 * s2
    bm00 = _bf16(m00)
    bm01 = _bf16(m01)
    bm02 = _bf16(m02)
    bm10 = _bf16(m10)
    bm11 = _bf16(m11)
    bm12 = _bf16(m12)
    bm20 = _bf16(m20)
    bm21 = _bf16(m21)
    bm22 = _bf16(m22)
    s_00 = bm00 * bm00 + bm01 * bm01 + bm02 * bm02
    s_01 = bm00 * bm10 + bm01 * bm11 + bm02 * bm12
    s_02 = bm00 * bm20 + bm01 * bm21 + bm02 * bm22
    s_11 = bm10 * bm10 + bm11 * bm11 + bm12 * bm12
    s_12 = bm10 * bm20 + bm11 * bm21 + bm12 * bm22
    s_22 = bm20 * bm20 + bm21 * bm21 + bm22 * bm22
    br = [_bf16(p[i]) for i in range(9)]
    bs00 = _bf16(s_00)
    bs01 = _bf16(s_01)
    bs02 = _bf16(s_02)
    bs11 = _bf16(s_11)
    bs12 = _bf16(s_12)
    bs22 = _bf16(s_22)
    a00 = br[0] * bs00 + br[1] * bs01 + br[2] * bs02
    a01 = br[0] * bs01 + br[1] * bs11 + br[2] * bs12
    a02 = br[0] * bs02 + br[1] * bs12 + br[2] * bs22
    a10 = br[3] * bs00 + br[4] * bs01 + br[5] * bs02
    a11 = br[3] * bs01 + br[4] * bs11 + br[5] * bs12
    a12 = br[3] * bs02 + br[4] * bs12 + br[5] * bs22
    a20 = br[6] * bs00 + br[7] * bs01 + br[8] * bs02
    a21 = br[6] * bs01 + br[7] * bs11 + br[8] * bs12
    a22 = br[6] * bs02 + br[7] * bs12 + br[8] * bs22
    ba00 = _bf16(a00)
    ba01 = _bf16(a01)
    ba02 = _bf16(a02)
    ba10 = _bf16(a10)
    ba11 = _bf16(a11)
    ba12 = _bf16(a12)
    ba20 = _bf16(a20)
    ba21 = _bf16(a21)
    ba22 = _bf16(a22)
    c00 = ba00 * br[0] + ba01 * br[1] + ba02 * br[2]
    c01 = ba00 * br[3] + ba01 * br[4] + ba02 * br[5]
    c02 = ba00 * br[6] + ba01 * br[7] + ba02 * br[8]
    c10 = ba10 * br[0] + ba11 * br[1] + ba12 * br[2]
    c11 = ba10 * br[3] + ba11 * br[4] + ba12 * br[5]
    c12 = ba10 * br[6] + ba11 * br[7] + ba12 * br[8]
    c20 = ba20 * br[0] + ba21 * br[1] + ba22 * br[2]
    c21 = ba20 * br[3] + ba21 * br[4] + ba22 * br[5]
    c22 = ba20 * br[6] + ba21 * br[7] + ba22 * br[8]
    ja = fx / d_safe
    jb = -fx * xc / (d_safe * d_safe)
    jc = fy / d_safe
    je = -fy * yc / (d_safe * d_safe)
    bja = _bf16(ja)
    bjb = _bf16(jb)
    bjc = _bf16(jc)
    bje = _bf16(je)
    t100 = bja * _bf16(c00) + bjb * _bf16(c20)
    t101 = bja * _bf16(c01) + bjb * _bf16(c21)
    t102 = bja * _bf16(c02) + bjb * _bf16(c22)
    t110 = bjc * _bf16(c10) + bje * _bf16(c20)
    t111 = bjc * _bf16(c11) + bje * _bf16(c21)
    t112 = bjc * _bf16(c12) + bje * _bf16(c22)
    cov00 = _bf16(t100) * bja + _bf16(t102) * bjb
    cov01 = _bf16(t101) * bjc + _bf16(t102) * bje
    cov10 = _bf16(t110) * bja + _bf16(t112) * bjb
    cov11 = _bf16(t111) * bjc + _bf16(t112) * bje

    zero = jnp.zeros_like(u)
    attrs_ref[0] = jnp.where(mask, u, zero)
    attrs_ref[1] = jnp.where(mask, v, zero)
    attrs_ref[2] = jnp.where(mask, xc, zero)
    attrs_ref[3] = jnp.where(mask, yc, zero)
    attrs_ref[4] = jnp.where(mask, d, zero)
    attrs_ref[5] = jnp.where(mask, cov00, zero)
    attrs_ref[6] = jnp.where(mask, cov01, zero)
    attrs_ref[7] = jnp.where(mask, cov10, zero)
    attrs_ref[8] = jnp.where(mask, cov11, zero)

    depth_key = (d * _DEPTH_TO_SORT_KEY_SCALE).astype(jnp.int32)
    tile_f = jnp.floor(u * (1.0 / 16.0)) + jnp.floor(v * (1.0 / 16.0)) * tiles_per_row
    key = (tile_f.astype(jnp.int32) << _KEY_DEPTH_BITS) + depth_key
    key_ref[...] = jnp.where(mask, key, jnp.int32(2 ** 30))


def _compute_attrs_and_key(pc3, ft3, params, grid):
    return pl.pallas_call(
        _attrs_key_body,
        grid=(grid,),
        in_specs=[
            pl.BlockSpec(memory_space=pltpu.SMEM),
            pl.BlockSpec((3, _RB, _C), lambda i: (0, i, 0)),
            pl.BlockSpec((7, _RB, _C), lambda i: (0, i, 0)),
        ],
        out_specs=[
            pl.BlockSpec((9, _RB, _C), lambda i: (0, i, 0)),
            pl.BlockSpec((_RB, _C), lambda i: (i, 0)),
        ],
        out_shape=[
            jax.ShapeDtypeStruct((9, grid * _RB, _C), jnp.float32),
            jax.ShapeDtypeStruct((grid * _RB, _C), jnp.int32),
        ],
        interpret=_INTERPRET,
    )(params, pc3, ft3)


def kernel(point_cloud, point_cloud_features, camera_intrinsics,
           T_pointcloud_camera, camera_width, camera_height):
    n = point_cloud.shape[0]

    T_camera_pointcloud = jnp.linalg.inv(T_pointcloud_camera)
    rcw = T_camera_pointcloud[:3, :3]
    tcw = T_camera_pointcloud[:3, 3]
    width_f = jnp.asarray(camera_width, jnp.float32)
    height_f = jnp.asarray(camera_height, jnp.float32)
    tiles_per_row_f = jnp.asarray(camera_width // 16, jnp.float32)
    params = jnp.concatenate([
        rcw.reshape(9), tcw.reshape(3),
        jnp.stack([camera_intrinsics[0, 0], camera_intrinsics[1, 1],
                   camera_intrinsics[0, 2], camera_intrinsics[1, 2],
                   width_f, height_f, tiles_per_row_f]),
    ]).astype(jnp.float32)

    blk = _RB * _C
    grid = -(-n // blk)
    n_pad = grid * blk
    xyz_cam = point_cloud @ rcw.T + tcw
    pc_t = xyz_cam.T
    ft_t = point_cloud_features[:, :7].T
    if n_pad != n:
        pc_t = jnp.pad(pc_t, ((0, 0), (0, n_pad - n)))
        ft_t = jnp.pad(ft_t, ((0, 0), (0, n_pad - n)))
    pc3 = pc_t.reshape(3, grid * _RB, _C)
    ft3 = ft_t.reshape(7, grid * _RB, _C)

    attrs9, key2 = _compute_attrs_and_key(pc3, ft3, params, grid)
    key = key2.reshape(n_pad)[:n]
    attrs_rows = attrs9.reshape(9, n_pad)[:, :n].T

    iota = lax.iota(jnp.int32, n)
    _, perm = lax.sort((key, iota), num_keys=1, is_stable=True)
    return jnp.take(attrs_rows, perm, axis=0, mode="clip")


# speedup vs baseline: 9.4503x; 1.0005x over previous
import functools

import jax
import jax.numpy as jnp
from jax import lax
from jax.experimental import pallas as pl
from jax.experimental.pallas import tpu as pltpu

_NEAR_PLANE = 0.8
_FAR_PLANE = 1000.0
_DEPTH_TO_SORT_KEY_SCALE = 100.0
_KEY_DEPTH_BITS = 17
_INTERPRET = False

_C = 512
_RB = 64


def _bf16(v):
    return v.astype(jnp.bfloat16).astype(jnp.float32)


def _attrs_key_body(params_ref, pc_ref, ft_ref, attrs_ref, key_ref):
    p = params_ref
    xc = pc_ref[0]
    yc = pc_ref[1]
    d = pc_ref[2]
    fx = p[12]
    fy = p[13]
    cx = p[14]
    cy = p[15]
    width = p[16]
    height = p[17]
    tiles_per_row = p[18]
    sentinel_key = p[19].astype(jnp.int32) << _KEY_DEPTH_BITS
    d_safe = jnp.where(jnp.abs(d) > 1e-6, d, 1e-6)
    u = fx * xc / d_safe + cx
    v = fy * yc / d_safe + cy
    mask = ((d > _NEAR_PLANE) & (d < _FAR_PLANE)
            & (u >= 0) & (u < width) & (v >= 0) & (v < height))

    qx = ft_ref[0]
    qy = ft_ref[1]
    qz = ft_ref[2]
    qw = ft_ref[3]
    inv_qn = 1.0 / (jnp.sqrt(qx * qx + qy * qy + qz * qz + qw * qw) + 1e-8)
    qx = qx * inv_qn
    qy = qy * inv_qn
    qz = qz * inv_qn
    qw = qw * inv_qn
    r00 = 1.0 - 2.0 * (qy * qy + qz * qz)
    r01 = 2.0 * (qx * qy - qw * qz)
    r02 = 2.0 * (qx * qz + qw * qy)
    r10 = 2.0 * (qx * qy + qw * qz)
    r11 = 1.0 - 2.0 * (qx * qx + qz * qz)
    r12 = 2.0 * (qy * qz - qw * qx)
    r20 = 2.0 * (qx * qz - qw * qy)
    r21 = 2.0 * (qy * qz + qw * qx)
    r22 = 1.0 - 2.0 * (qx * qx + qy * qy)
    s0 = jnp.exp(ft_ref[4])
    s1 = jnp.exp(ft_ref[5])
    s2 = jnp.exp(ft_ref[6])
    m00 = r00 * s0
    m01 = r01 * s1
    m02 = r02 * s2
    m10 = r10 * s0
    m11 = r11 * s1
    m12 = r12 * s2
    m20 = r20 * s0
    m21 = r21 * s1
    m22 = r22 * s2
    bm00 = _bf16(m00)
    bm01 = _bf16(m01)
    bm02 = _bf16(m02)
    bm10 = _bf16(m10)
    bm11 = _bf16(m11)
    bm12 = _bf16(m12)
    bm20 = _bf16(m20)
    bm21 = _bf16(m21)
    bm22 = _bf16(m22)
    s_00 = bm00 * bm00 + bm01 * bm01 + bm02 * bm02
    s_01 = bm00 * bm10 + bm01 * bm11 + bm02 * bm12
    s_02 = bm00 * bm20 + bm01 * bm21 + bm02 * bm22
    s_11 = bm10 * bm10 + bm11 * bm11 + bm12 * bm12
    s_12 = bm10 * bm20 + bm11 * bm21 + bm12 * bm22
    s_22 = bm20 * bm20 + bm21 * bm21 + bm22 * bm22
    br = [_bf16(p[i]) for i in range(9)]
    bs00 = _bf16(s_00)
    bs01 = _bf16(s_01)
    bs02 = _bf16(s_02)
    bs11 = _bf16(s_11)
    bs12 = _bf16(s_12)
    bs22 = _bf16(s_22)
    a00 = br[0] * bs00 + br[1] * bs01 + br[2] * bs02
    a01 = br[0] * bs01 + br[1] * bs11 + br[2] * bs12
    a02 = br[0] * bs02 + br[1] * bs12 + br[2] * bs22
    a10 = br[3] * bs00 + br[4] * bs01 + br[5] * bs02
    a11 = br[3] * bs01 + br[4] * bs11 + br[5] * bs12
    a12 = br[3] * bs02 + br[4] * bs12 + br[5] * bs22
    a20 = br[6] * bs00 + br[7] * bs01 + br[8] * bs02
    a21 = br[6] * bs01 + br[7] * bs11 + br[8] * bs12
    a22 = br[6] * bs02 + br[7] * bs12 + br[8] * bs22
    ba00 = _bf16(a00)
    ba01 = _bf16(a01)
    ba02 = _bf16(a02)
    ba10 = _bf16(a10)
    ba11 = _bf16(a11)
    ba12 = _bf16(a12)
    ba20 = _bf16(a20)
    ba21 = _bf16(a21)
    ba22 = _bf16(a22)
    c00 = ba00 * br[0] + ba01 * br[1] + ba02 * br[2]
    c01 = ba00 * br[3] + ba01 * br[4] + ba02 * br[5]
    c02 = ba00 * br[6] + ba01 * br[7] + ba02 * br[8]
    c10 = ba10 * br[0] + ba11 * br[1] + ba12 * br[2]
    c11 = ba10 * br[3] + ba11 * br[4] + ba12 * br[5]
    c12 = ba10 * br[6] + ba11 * br[7] + ba12 * br[8]
    c20 = ba20 * br[0] + ba21 * br[1] + ba22 * br[2]
    c21 = ba20 * br[3] + ba21 * br[4] + ba22 * br[5]
    c22 = ba20 * br[6] + ba21 * br[7] + ba22 * br[8]
    ja = fx / d_safe
    jb = -fx * xc / (d_safe * d_safe)
    jc = fy / d_safe
    je = -fy * yc / (d_safe * d_safe)
    bja = _bf16(ja)
    bjb = _bf16(jb)
    bjc = _bf16(jc)
    bje = _bf16(je)
    t100 = bja * _bf16(c00) + bjb * _bf16(c20)
    t101 = bja * _bf16(c01) + bjb * _bf16(c21)
    t102 = bja * _bf16(c02) + bjb * _bf16(c22)
    t110 = bjc * _bf16(c10) + bje * _bf16(c20)
    t111 = bjc * _bf16(c11) + bje * _bf16(c21)
    t112 = bjc * _bf16(c12) + bje * _bf16(c22)
    cov00 = _bf16(t100) * bja + _bf16(t102) * bjb
    cov01 = _bf16(t101) * bjc + _bf16(t102) * bje
    cov10 = _bf16(t110) * bja + _bf16(t112) * bjb
    cov11 = _bf16(t111) * bjc + _bf16(t112) * bje

    zero = jnp.zeros_like(u)
    attrs_ref[0] = jnp.where(mask, u, zero)
    attrs_ref[1] = jnp.where(mask, v, zero)
    attrs_ref[2] = jnp.where(mask, xc, zero)
    attrs_ref[3] = jnp.where(mask, yc, zero)
    attrs_ref[4] = jnp.where(mask, d, zero)
    attrs_ref[5] = jnp.where(mask, cov00, zero)
    attrs_ref[6] = jnp.where(mask, cov01, zero)
    attrs_ref[7] = jnp.where(mask, cov10, zero)
    attrs_ref[8] = jnp.where(mask, cov11, zero)

    depth_key = (d * _DEPTH_TO_SORT_KEY_SCALE).astype(jnp.int32)
    tile_f = jnp.floor(u * (1.0 / 16.0)) + jnp.floor(v * (1.0 / 16.0)) * tiles_per_row
    key = (tile_f.astype(jnp.int32) << _KEY_DEPTH_BITS) + depth_key
    key_ref[...] = jnp.where(mask, key, sentinel_key)


def _compute_attrs_and_key(pc3, ft3, params, grid):
    return pl.pallas_call(
        _attrs_key_body,
        grid=(grid,),
        in_specs=[
            pl.BlockSpec(memory_space=pltpu.SMEM),
            pl.BlockSpec((3, _RB, _C), lambda i: (0, i, 0)),
            pl.BlockSpec((7, _RB, _C), lambda i: (0, i, 0)),
        ],
        out_specs=[
            pl.BlockSpec((9, _RB, _C), lambda i: (0, i, 0)),
            pl.BlockSpec((_RB, _C), lambda i: (i, 0)),
        ],
        out_shape=[
            jax.ShapeDtypeStruct((9, grid * _RB, _C), jnp.float32),
            jax.ShapeDtypeStruct((grid * _RB, _C), jnp.int32),
        ],
        interpret=_INTERPRET,
    )(params, pc3, ft3)


def kernel(point_cloud, point_cloud_features, camera_intrinsics,
           T_pointcloud_camera, camera_width, camera_height):
    n = point_cloud.shape[0]

    T_camera_pointcloud = jnp.linalg.inv(T_pointcloud_camera)
    rcw = T_camera_pointcloud[:3, :3]
    tcw = T_camera_pointcloud[:3, 3]
    width_f = jnp.asarray(camera_width, jnp.float32)
    height_f = jnp.asarray(camera_height, jnp.float32)
    tiles_per_row_f = jnp.asarray(camera_width // 16, jnp.float32)
    params = jnp.concatenate([
        rcw.reshape(9), tcw.reshape(3),
        jnp.stack([camera_intrinsics[0, 0], camera_intrinsics[1, 1],
                   camera_intrinsics[0, 2], camera_intrinsics[1, 2],
                   width_f, height_f, tiles_per_row_f,
                   jnp.asarray((camera_width // 16) * ((camera_height + 15) // 16),
                               jnp.float32)]),
    ]).astype(jnp.float32)

    blk = _RB * _C
    grid = -(-n // blk)
    n_pad = grid * blk
    xyz_cam = point_cloud @ rcw.T + tcw
    pc_t = xyz_cam.T
    ft_t = point_cloud_features[:, :7].T
    if n_pad != n:
        pc_t = jnp.pad(pc_t, ((0, 0), (0, n_pad - n)))
        ft_t = jnp.pad(ft_t, ((0, 0), (0, n_pad - n)))
    pc3 = pc_t.reshape(3, grid * _RB, _C)
    ft3 = ft_t.reshape(7, grid * _RB, _C)

    attrs9, key2 = _compute_attrs_and_key(pc3, ft3, params, grid)
    key = key2.reshape(n_pad)[:n]
    attrs_rows = attrs9.reshape(9, n_pad)[:, :n].T

    iota = lax.iota(jnp.int32, n)
    _, perm = lax.sort((key, iota), num_keys=1, is_stable=True)
    return jnp.take(attrs_rows, perm, axis=0, mode="clip")


# speedup vs baseline: 11.2483x; 1.1902x over previous
import functools

import jax
import jax.numpy as jnp
from jax import lax
from jax.experimental import pallas as pl
from jax.experimental.pallas import tpu as pltpu
from jax.experimental.pallas import tpu_sc as plsc

_NEAR_PLANE = 0.8
_FAR_PLANE = 1000.0
_DEPTH_TO_SORT_KEY_SCALE = 100.0
_KEY_DEPTH_BITS = 17
_INTERPRET = False

_C = 512
_RB = 64


def _bf16(v):
    return v.astype(jnp.bfloat16).astype(jnp.float32)


def _attrs_key_body(params_ref, pc_ref, ft_ref, attrs_ref, key_ref):
    p = params_ref
    xc = pc_ref[0]
    yc = pc_ref[1]
    d = pc_ref[2]
    fx = p[12]
    fy = p[13]
    cx = p[14]
    cy = p[15]
    width = p[16]
    height = p[17]
    tiles_per_row = p[18]
    sentinel_key = p[19].astype(jnp.int32) << _KEY_DEPTH_BITS
    d_safe = jnp.where(jnp.abs(d) > 1e-6, d, 1e-6)
    u = fx * xc / d_safe + cx
    v = fy * yc / d_safe + cy
    mask = ((d > _NEAR_PLANE) & (d < _FAR_PLANE)
            & (u >= 0) & (u < width) & (v >= 0) & (v < height))

    qx = ft_ref[0]
    qy = ft_ref[1]
    qz = ft_ref[2]
    qw = ft_ref[3]
    inv_qn = 1.0 / (jnp.sqrt(qx * qx + qy * qy + qz * qz + qw * qw) + 1e-8)
    qx = qx * inv_qn
    qy = qy * inv_qn
    qz = qz * inv_qn
    qw = qw * inv_qn
    r00 = 1.0 - 2.0 * (qy * qy + qz * qz)
    r01 = 2.0 * (qx * qy - qw * qz)
    r02 = 2.0 * (qx * qz + qw * qy)
    r10 = 2.0 * (qx * qy + qw * qz)
    r11 = 1.0 - 2.0 * (qx * qx + qz * qz)
    r12 = 2.0 * (qy * qz - qw * qx)
    r20 = 2.0 * (qx * qz - qw * qy)
    r21 = 2.0 * (qy * qz + qw * qx)
    r22 = 1.0 - 2.0 * (qx * qx + qy * qy)
    s0 = jnp.exp(ft_ref[4])
    s1 = jnp.exp(ft_ref[5])
    s2 = jnp.exp(ft_ref[6])
    m00 = r00 * s0
    m01 = r01 * s1
    m02 = r02 * s2
    m10 = r10 * s0
    m11 = r11 * s1
    m12 = r12 * s2
    m20 = r20 * s0
    m21 = r21 * s1
    m22 = r22 * s2
    bm00 = _bf16(m00)
    bm01 = _bf16(m01)
    bm02 = _bf16(m02)
    bm10 = _bf16(m10)
    bm11 = _bf16(m11)
    bm12 = _bf16(m12)
    bm20 = _bf16(m20)
    bm21 = _bf16(m21)
    bm22 = _bf16(m22)
    s_00 = bm00 * bm00 + bm01 * bm01 + bm02 * bm02
    s_01 = bm00 * bm10 + bm01 * bm11 + bm02 * bm12
    s_02 = bm00 * bm20 + bm01 * bm21 + bm02 * bm22
    s_11 = bm10 * bm10 + bm11 * bm11 + bm12 * bm12
    s_12 = bm10 * bm20 + bm11 * bm21 + bm12 * bm22
    s_22 = bm20 * bm20 + bm21 * bm21 + bm22 * bm22
    br = [_bf16(p[i]) for i in range(9)]
    bs00 = _bf16(s_00)
    bs01 = _bf16(s_01)
    bs02 = _bf16(s_02)
    bs11 = _bf16(s_11)
    bs12 = _bf16(s_12)
    bs22 = _bf16(s_22)
    a00 = br[0] * bs00 + br[1] * bs01 + br[2] * bs02
    a01 = br[0] * bs01 + br[1] * bs11 + br[2] * bs12
    a02 = br[0] * bs02 + br[1] * bs12 + br[2] * bs22
    a10 = br[3] * bs00 + br[4] * bs01 + br[5] * bs02
    a11 = br[3] * bs01 + br[4] * bs11 + br[5] * bs12
    a12 = br[3] * bs02 + br[4] * bs12 + br[5] * bs22
    a20 = br[6] * bs00 + br[7] * bs01 + br[8] * bs02
    a21 = br[6] * bs01 + br[7] * bs11 + br[8] * bs12
    a22 = br[6] * bs02 + br[7] * bs12 + br[8] * bs22
    ba00 = _bf16(a00)
    ba01 = _bf16(a01)
    ba02 = _bf16(a02)
    ba10 = _bf16(a10)
    ba11 = _bf16(a11)
    ba12 = _bf16(a12)
    ba20 = _bf16(a20)
    ba21 = _bf16(a21)
    ba22 = _bf16(a22)
    c00 = ba00 * br[0] + ba01 * br[1] + ba02 * br[2]
    c01 = ba00 * br[3] + ba01 * br[4] + ba02 * br[5]
    c02 = ba00 * br[6] + ba01 * br[7] + ba02 * br[8]
    c10 = ba10 * br[0] + ba11 * br[1] + ba12 * br[2]
    c11 = ba10 * br[3] + ba11 * br[4] + ba12 * br[5]
    c12 = ba10 * br[6] + ba11 * br[7] + ba12 * br[8]
    c20 = ba20 * br[0] + ba21 * br[1] + ba22 * br[2]
    c21 = ba20 * br[3] + ba21 * br[4] + ba22 * br[5]
    c22 = ba20 * br[6] + ba21 * br[7] + ba22 * br[8]
    ja = fx / d_safe
    jb = -fx * xc / (d_safe * d_safe)
    jc = fy / d_safe
    je = -fy * yc / (d_safe * d_safe)
    bja = _bf16(ja)
    bjb = _bf16(jb)
    bjc = _bf16(jc)
    bje = _bf16(je)
    t100 = bja * _bf16(c00) + bjb * _bf16(c20)
    t101 = bja * _bf16(c01) + bjb * _bf16(c21)
    t102 = bja * _bf16(c02) + bjb * _bf16(c22)
    t110 = bjc * _bf16(c10) + bje * _bf16(c20)
    t111 = bjc * _bf16(c11) + bje * _bf16(c21)
    t112 = bjc * _bf16(c12) + bje * _bf16(c22)
    cov00 = _bf16(t100) * bja + _bf16(t102) * bjb
    cov01 = _bf16(t101) * bjc + _bf16(t102) * bje
    cov10 = _bf16(t110) * bja + _bf16(t112) * bjb
    cov11 = _bf16(t111) * bjc + _bf16(t112) * bje

    zero = jnp.zeros_like(u)
    attrs_ref[0] = jnp.where(mask, u, zero)
    attrs_ref[1] = jnp.where(mask, v, zero)
    attrs_ref[2] = jnp.where(mask, xc, zero)
    attrs_ref[3] = jnp.where(mask, yc, zero)
    attrs_ref[4] = jnp.where(mask, d, zero)
    attrs_ref[5] = jnp.where(mask, cov00, zero)
    attrs_ref[6] = jnp.where(mask, cov01, zero)
    attrs_ref[7] = jnp.where(mask, cov10, zero)
    attrs_ref[8] = jnp.where(mask, cov11, zero)
    attrs_ref[9] = zero
    attrs_ref[10] = zero
    attrs_ref[11] = zero
    attrs_ref[12] = zero
    attrs_ref[13] = zero
    attrs_ref[14] = zero
    attrs_ref[15] = zero

    depth_key = (d * _DEPTH_TO_SORT_KEY_SCALE).astype(jnp.int32)
    tile_f = jnp.floor(u * (1.0 / 16.0)) + jnp.floor(v * (1.0 / 16.0)) * tiles_per_row
    key = (tile_f.astype(jnp.int32) << _KEY_DEPTH_BITS) + depth_key
    key_ref[...] = jnp.where(mask, key, sentinel_key)


def _compute_attrs_and_key(pc3, ft3, params, grid):
    return pl.pallas_call(
        _attrs_key_body,
        grid=(grid,),
        in_specs=[
            pl.BlockSpec(memory_space=pltpu.SMEM),
            pl.BlockSpec((3, _RB, _C), lambda i: (0, i, 0)),
            pl.BlockSpec((7, _RB, _C), lambda i: (0, i, 0)),
        ],
        out_specs=[
            pl.BlockSpec((16, _RB, _C), lambda i: (0, i, 0)),
            pl.BlockSpec((_RB, _C), lambda i: (i, 0)),
        ],
        out_shape=[
            jax.ShapeDtypeStruct((16, grid * _RB, _C), jnp.float32),
            jax.ShapeDtypeStruct((grid * _RB, _C), jnp.int32),
        ],
        interpret=_INTERPRET,
    )(params, pc3, ft3)


_NW = 32
_CH = 128
_KB = 8


def _make_sc_gather(n_pad):
    rows_per_w = n_pad // _NW
    n_chunks = rows_per_w // _CH
    n_groups = n_chunks // _KB
    mesh = plsc.VectorSubcoreMesh(core_axis_name="c", subcore_axis_name="s")

    @functools.partial(
        pl.kernel, mesh=mesh,
        out_type=jax.ShapeDtypeStruct((n_pad, 9), jnp.float32),
        scratch_types=[
            pltpu.VMEM((rows_per_w,), jnp.int32),
            pltpu.VMEM((_KB, _CH, 9), jnp.float32),
            pltpu.SemaphoreType.DMA,
        ],
    )
    def sc_gather(attrs_hbm, perm_hbm, out_hbm, idx_v, rows_v, sem):
        wid = lax.axis_index("s") * 2 + lax.axis_index("c")
        base = wid * rows_per_w
        pltpu.sync_copy(perm_hbm.at[pl.ds(base, rows_per_w)], idx_v)

        def group(g, _):
            copies = []
            for b in range(_KB):
                c = g * _KB + b
                copies.append(pltpu.async_copy(
                    attrs_hbm.at[idx_v.at[pl.ds(c * _CH, _CH)]],
                    rows_v.at[b], sem))
            for b in range(_KB):
                copies[b].wait()
            for b in range(_KB):
                c = g * _KB + b
                pltpu.sync_copy(rows_v.at[b],
                                out_hbm.at[pl.ds(base + c * _CH, _CH)])
            return _

        lax.fori_loop(0, n_groups, group, None)

    return sc_gather


def kernel(point_cloud, point_cloud_features, camera_intrinsics,
           T_pointcloud_camera, camera_width, camera_height):
    n = point_cloud.shape[0]

    T_camera_pointcloud = jnp.linalg.inv(T_pointcloud_camera)
    rcw = T_camera_pointcloud[:3, :3]
    tcw = T_camera_pointcloud[:3, 3]
    width_f = jnp.asarray(camera_width, jnp.float32)
    height_f = jnp.asarray(camera_height, jnp.float32)
    tiles_per_row_f = jnp.asarray(camera_width // 16, jnp.float32)
    params = jnp.concatenate([
        rcw.reshape(9), tcw.reshape(3),
        jnp.stack([camera_intrinsics[0, 0], camera_intrinsics[1, 1],
                   camera_intrinsics[0, 2], camera_intrinsics[1, 2],
                   width_f, height_f, tiles_per_row_f,
                   jnp.asarray((camera_width // 16) * ((camera_height + 15) // 16),
                               jnp.float32)]),
    ]).astype(jnp.float32)

    blk = _RB * _C
    grid = -(-n // blk)
    n_pad = grid * blk
    xyz_cam = point_cloud @ rcw.T + tcw
    pc_t = xyz_cam.T
    ft_t = point_cloud_features[:, :7].T
    if n_pad != n:
        pc_t = jnp.pad(pc_t, ((0, 0), (0, n_pad - n)))
        ft_t = jnp.pad(ft_t, ((0, 0), (0, n_pad - n)))
    pc3 = pc_t.reshape(3, grid * _RB, _C)
    ft3 = ft_t.reshape(7, grid * _RB, _C)

    attrs9, key2 = _compute_attrs_and_key(pc3, ft3, params, grid)
    key = key2.reshape(n_pad)
    attrs_rows = attrs9.reshape(16, n_pad).T

    iota = lax.iota(jnp.int32, n_pad)
    _, perm = lax.sort((key, iota), num_keys=1, is_stable=True)
    gathered = jnp.take(attrs_rows, perm, axis=0, mode="clip")
    return gathered[:n, :9]


# speedup vs baseline: 11.2500x; 1.0002x over previous
import functools

import jax
import jax.numpy as jnp
from jax import lax
from jax.experimental import pallas as pl
from jax.experimental.pallas import tpu as pltpu
from jax.experimental.pallas import tpu_sc as plsc
from jax.experimental import compute_on

_NEAR_PLANE = 0.8
_FAR_PLANE = 1000.0
_DEPTH_TO_SORT_KEY_SCALE = 100.0
_KEY_DEPTH_BITS = 17
_INTERPRET = False

_C = 512
_RB = 64


def _bf16(v):
    return v.astype(jnp.bfloat16).astype(jnp.float32)


def _attrs_key_body(params_ref, pc_ref, ft_ref, attrs_ref, key_ref):
    p = params_ref
    xc = pc_ref[0]
    yc = pc_ref[1]
    d = pc_ref[2]
    fx = p[12]
    fy = p[13]
    cx = p[14]
    cy = p[15]
    width = p[16]
    height = p[17]
    tiles_per_row = p[18]
    sentinel_key = p[19].astype(jnp.int32) << _KEY_DEPTH_BITS
    d_safe = jnp.where(jnp.abs(d) > 1e-6, d, 1e-6)
    u = fx * xc / d_safe + cx
    v = fy * yc / d_safe + cy
    mask = ((d > _NEAR_PLANE) & (d < _FAR_PLANE)
            & (u >= 0) & (u < width) & (v >= 0) & (v < height))

    qx = ft_ref[0]
    qy = ft_ref[1]
    qz = ft_ref[2]
    qw = ft_ref[3]
    inv_qn = 1.0 / (jnp.sqrt(qx * qx + qy * qy + qz * qz + qw * qw) + 1e-8)
    qx = qx * inv_qn
    qy = qy * inv_qn
    qz = qz * inv_qn
    qw = qw * inv_qn
    r00 = 1.0 - 2.0 * (qy * qy + qz * qz)
    r01 = 2.0 * (qx * qy - qw * qz)
    r02 = 2.0 * (qx * qz + qw * qy)
    r10 = 2.0 * (qx * qy + qw * qz)
    r11 = 1.0 - 2.0 * (qx * qx + qz * qz)
    r12 = 2.0 * (qy * qz - qw * qx)
    r20 = 2.0 * (qx * qz - qw * qy)
    r21 = 2.0 * (qy * qz + qw * qx)
    r22 = 1.0 - 2.0 * (qx * qx + qy * qy)
    s0 = jnp.exp(ft_ref[4])
    s1 = jnp.exp(ft_ref[5])
    s2 = jnp.exp(ft_ref[6])
    m00 = r00 * s0
    m01 = r01 * s1
    m02 = r02 * s2
    m10 = r10 * s0
    m11 = r11 * s1
    m12 = r12 * s2
    m20 = r20 * s0
    m21 = r21 * s1
    m22 = r22 * s2
    bm00 = _bf16(m00)
    bm01 = _bf16(m01)
    bm02 = _bf16(m02)
    bm10 = _bf16(m10)
    bm11 = _bf16(m11)
    bm12 = _bf16(m12)
    bm20 = _bf16(m20)
    bm21 = _bf16(m21)
    bm22 = _bf16(m22)
    s_00 = bm00 * bm00 + bm01 * bm01 + bm02 * bm02
    s_01 = bm00 * bm10 + bm01 * bm11 + bm02 * bm12
    s_02 = bm00 * bm20 + bm01 * bm21 + bm02 * bm22
    s_11 = bm10 * bm10 + bm11 * bm11 + bm12 * bm12
    s_12 = bm10 * bm20 + bm11 * bm21 + bm12 * bm22
    s_22 = bm20 * bm20 + bm21 * bm21 + bm22 * bm22
    br = [_bf16(p[i]) for i in range(9)]
    bs00 = _bf16(s_00)
    bs01 = _bf16(s_01)
    bs02 = _bf16(s_02)
    bs11 = _bf16(s_11)
    bs12 = _bf16(s_12)
    bs22 = _bf16(s_22)
    a00 = br[0] * bs00 + br[1] * bs01 + br[2] * bs02
    a01 = br[0] * bs01 + br[1] * bs11 + br[2] * bs12
    a02 = br[0] * bs02 + br[1] * bs12 + br[2] * bs22
    a10 = br[3] * bs00 + br[4] * bs01 + br[5] * bs02
    a11 = br[3] * bs01 + br[4] * bs11 + br[5] * bs12
    a12 = br[3] * bs02 + br[4] * bs12 + br[5] * bs22
    a20 = br[6] * bs00 + br[7] * bs01 + br[8] * bs02
    a21 = br[6] * bs01 + br[7] * bs11 + br[8] * bs12
    a22 = br[6] * bs02 + br[7] * bs12 + br[8] * bs22
    ba00 = _bf16(a00)
    ba01 = _bf16(a01)
    ba02 = _bf16(a02)
    ba10 = _bf16(a10)
    ba11 = _bf16(a11)
    ba12 = _bf16(a12)
    ba20 = _bf16(a20)
    ba21 = _bf16(a21)
    ba22 = _bf16(a22)
    c00 = ba00 * br[0] + ba01 * br[1] + ba02 * br[2]
    c01 = ba00 * br[3] + ba01 * br[4] + ba02 * br[5]
    c02 = ba00 * br[6] + ba01 * br[7] + ba02 * br[8]
    c10 = ba10 * br[0] + ba11 * br[1] + ba12 * br[2]
    c11 = ba10 * br[3] + ba11 * br[4] + ba12 * br[5]
    c12 = ba10 * br[6] + ba11 * br[7] + ba12 * br[8]
    c20 = ba20 * br[0] + ba21 * br[1] + ba22 * br[2]
    c21 = ba20 * br[3] + ba21 * br[4] + ba22 * br[5]
    c22 = ba20 * br[6] + ba21 * br[7] + ba22 * br[8]
    ja = fx / d_safe
    jb = -fx * xc / (d_safe * d_safe)
    jc = fy / d_safe
    je = -fy * yc / (d_safe * d_safe)
    bja = _bf16(ja)
    bjb = _bf16(jb)
    bjc = _bf16(jc)
    bje = _bf16(je)
    t100 = bja * _bf16(c00) + bjb * _bf16(c20)
    t101 = bja * _bf16(c01) + bjb * _bf16(c21)
    t102 = bja * _bf16(c02) + bjb * _bf16(c22)
    t110 = bjc * _bf16(c10) + bje * _bf16(c20)
    t111 = bjc * _bf16(c11) + bje * _bf16(c21)
    t112 = bjc * _bf16(c12) + bje * _bf16(c22)
    cov00 = _bf16(t100) * bja + _bf16(t102) * bjb
    cov01 = _bf16(t101) * bjc + _bf16(t102) * bje
    cov10 = _bf16(t110) * bja + _bf16(t112) * bjb
    cov11 = _bf16(t111) * bjc + _bf16(t112) * bje

    zero = jnp.zeros_like(u)
    attrs_ref[0] = jnp.where(mask, u, zero)
    attrs_ref[1] = jnp.where(mask, v, zero)
    attrs_ref[2] = jnp.where(mask, xc, zero)
    attrs_ref[3] = jnp.where(mask, yc, zero)
    attrs_ref[4] = jnp.where(mask, d, zero)
    attrs_ref[5] = jnp.where(mask, cov00, zero)
    attrs_ref[6] = jnp.where(mask, cov01, zero)
    attrs_ref[7] = jnp.where(mask, cov10, zero)
    attrs_ref[8] = jnp.where(mask, cov11, zero)
    attrs_ref[9] = zero
    attrs_ref[10] = zero
    attrs_ref[11] = zero
    attrs_ref[12] = zero
    attrs_ref[13] = zero
    attrs_ref[14] = zero
    attrs_ref[15] = zero

    depth_key = (d * _DEPTH_TO_SORT_KEY_SCALE).astype(jnp.int32)
    tile_f = jnp.floor(u * (1.0 / 16.0)) + jnp.floor(v * (1.0 / 16.0)) * tiles_per_row
    key = (tile_f.astype(jnp.int32) << _KEY_DEPTH_BITS) + depth_key
    key_ref[...] = jnp.where(mask, key, sentinel_key)


def _compute_attrs_and_key(pc3, ft3, params, grid):
    return pl.pallas_call(
        _attrs_key_body,
        grid=(grid,),
        in_specs=[
            pl.BlockSpec(memory_space=pltpu.SMEM),
            pl.BlockSpec((3, _RB, _C), lambda i: (0, i, 0)),
            pl.BlockSpec((7, _RB, _C), lambda i: (0, i, 0)),
        ],
        out_specs=[
            pl.BlockSpec((16, _RB, _C), lambda i: (0, i, 0)),
            pl.BlockSpec((_RB, _C), lambda i: (i, 0)),
        ],
        out_shape=[
            jax.ShapeDtypeStruct((16, grid * _RB, _C), jnp.float32),
            jax.ShapeDtypeStruct((grid * _RB, _C), jnp.int32),
        ],
        interpret=_INTERPRET,
    )(params, pc3, ft3)


_NW = 32
_CH = 128
_KB = 8


def _make_sc_gather(n_pad):
    rows_per_w = n_pad // _NW
    n_chunks = rows_per_w // _CH
    n_groups = n_chunks // _KB
    mesh = plsc.VectorSubcoreMesh(core_axis_name="c", subcore_axis_name="s")

    @functools.partial(
        pl.kernel, mesh=mesh,
        out_type=jax.ShapeDtypeStruct((n_pad, 9), jnp.float32),
        scratch_types=[
            pltpu.VMEM((rows_per_w,), jnp.int32),
            pltpu.VMEM((_KB, _CH, 9), jnp.float32),
            pltpu.SemaphoreType.DMA,
        ],
    )
    def sc_gather(attrs_hbm, perm_hbm, out_hbm, idx_v, rows_v, sem):
        wid = lax.axis_index("s") * 2 + lax.axis_index("c")
        base = wid * rows_per_w
        pltpu.sync_copy(perm_hbm.at[pl.ds(base, rows_per_w)], idx_v)

        def group(g, _):
            copies = []
            for b in range(_KB):
                c = g * _KB + b
                copies.append(pltpu.async_copy(
                    attrs_hbm.at[idx_v.at[pl.ds(c * _CH, _CH)]],
                    rows_v.at[b], sem))
            for b in range(_KB):
                copies[b].wait()
            for b in range(_KB):
                c = g * _KB + b
                pltpu.sync_copy(rows_v.at[b],
                                out_hbm.at[pl.ds(base + c * _CH, _CH)])
            return _

        lax.fori_loop(0, n_groups, group, None)

    return sc_gather


@functools.partial(compute_on.compute_on("tpu_sparsecore"), )
@jax.jit
def _sc_sort(key, iota):
    return lax.sort((key, iota), num_keys=1, is_stable=True)[1]


def kernel(point_cloud, point_cloud_features, camera_intrinsics,
           T_pointcloud_camera, camera_width, camera_height):
    n = point_cloud.shape[0]

    T_camera_pointcloud = jnp.linalg.inv(T_pointcloud_camera)
    rcw = T_camera_pointcloud[:3, :3]
    tcw = T_camera_pointcloud[:3, 3]
    width_f = jnp.asarray(camera_width, jnp.float32)
    height_f = jnp.asarray(camera_height, jnp.float32)
    tiles_per_row_f = jnp.asarray(camera_width // 16, jnp.float32)
    params = jnp.concatenate([
        rcw.reshape(9), tcw.reshape(3),
        jnp.stack([camera_intrinsics[0, 0], camera_intrinsics[1, 1],
                   camera_intrinsics[0, 2], camera_intrinsics[1, 2],
                   width_f, height_f, tiles_per_row_f,
                   jnp.asarray((camera_width // 16) * ((camera_height + 15) // 16),
                               jnp.float32)]),
    ]).astype(jnp.float32)

    blk = _RB * _C
    grid = -(-n // blk)
    n_pad = grid * blk
    xyz_cam = point_cloud @ rcw.T + tcw
    pc_t = xyz_cam.T
    ft_t = point_cloud_features[:, :7].T
    if n_pad != n:
        pc_t = jnp.pad(pc_t, ((0, 0), (0, n_pad - n)))
        ft_t = jnp.pad(ft_t, ((0, 0), (0, n_pad - n)))
    pc3 = pc_t.reshape(3, grid * _RB, _C)
    ft3 = ft_t.reshape(7, grid * _RB, _C)

    attrs9, key2 = _compute_attrs_and_key(pc3, ft3, params, grid)
    key = key2.reshape(n_pad)
    attrs_rows = attrs9.reshape(16, n_pad).T

    iota = lax.iota(jnp.int32, n_pad)
    perm = _sc_sort(key, iota)
    gathered = jnp.take(attrs_rows, perm, axis=0, mode="clip")
    return gathered[:n, :9]
